# baseline (speedup 1.0000x reference)
import jax
import jax.numpy as jnp
from jax import lax
from jax.experimental import pallas as pl
from jax.experimental.pallas import tpu as pltpu

_N = 32768
_B = 16
_BN = 8192
_NB = _N // _BN
_NEG = -1.0e30


def _dotg(a, b, dims):
  return lax.dot_general(a, b, (dims, ((), ())),
                         preferred_element_type=jnp.float32)


def _body(x_ref, ids_ref, idsf_ref, sl_ref, wu1_ref, bu1_ref, wu2_ref,
          bu2_ref, ww1_ref, bw1_ref, ww2_ref, bw2_ref,
          a_ref, wxt_ref, ct_ref, m_ref, s_ref, e_ref):
  i = pl.program_id(0)

  @pl.when(i == 0)
  def _init():
    ww1 = ww1_ref[...]
    w1_top = ww1[:150, :]
    w1_bot = ww1[150:, :]
    wxt_ref[...] = _dotg(w1_bot, wu2_ref[...], ((0,), (1,)))
    u1 = jnp.dot(sl_ref[...], wu1_ref[...],
                 preferred_element_type=jnp.float32) + bu1_ref[...]
    ct_ref[...] = (_dotg(w1_top, u1, ((0,), (1,)))
                   + _dotg(w1_bot, bu2_ref[...], ((0,), (1,)))
                   + bw1_ref[...])
    m_ref[...] = jnp.full((_B, 1), _NEG, jnp.float32)
    s_ref[...] = jnp.zeros((_B, 1), jnp.float32)

  @pl.when(i < _NB)
  def _scores():
    ids = ids_ref[...]
    ohb = ids == lax.broadcasted_iota(jnp.int32, (_B, 1), 0)
    ohf = ohb.astype(jnp.float32)
    x = x_ref[...]
    et = jnp.zeros((1, _BN), jnp.float32)
    for r0, r1 in ((0, 40), (40, 80), (80, 120), (120, 150)):
      pre = _dotg(wxt_ref[r0:r1, :], x, ((1,), (1,)))
      cg = _dotg(ct_ref[r0:r1, :], ohf, ((1,), (0,)))
      t = jnp.tanh(pre + cg)
      et = et + _dotg(ww2_ref[r0:r1, :], t, ((0,), (0,)))
    et = et + bw2_ref[...]
    e_ref[pl.ds(i, 1), :] = et
    m_part = jnp.max(jnp.where(ohb, et, _NEG), axis=1, keepdims=True)
    s_part = jnp.sum(jnp.where(ohb, jnp.exp(et), 0.0), axis=1, keepdims=True)
    m_ref[...] = jnp.maximum(m_ref[...], m_part)
    s_ref[...] = s_ref[...] + s_part

  @pl.when(i == _NB)
  def _normalize():
    m = jnp.maximum(m_ref[...], -80.0)
    em = jnp.exp(-m)
    s = s_ref[...]
    q = jnp.where(s > 0.0, em / (em * s + 1e-16), 0.0)
    iota_b = lax.broadcasted_iota(jnp.int32, (_B, 1), 0)
    for j in range(_NB):
      ids_j = idsf_ref[0:1, j * _BN:(j + 1) * _BN]
      ohf = (ids_j == iota_b).astype(jnp.float32)
      qg = _dotg(q, ohf, ((0,), (0,)))
      a_ref[0:1, j * _BN:(j + 1) * _BN] = (
          jnp.exp(e_ref[j:j + 1, :]) * qg)


def kernel(x_t, x_t_batch, smile_latent, W_U1, b_U1, W_U2, b_U2,
           W_W1, b_W1, W_W2, b_W2):
  ids = x_t_batch.astype(jnp.int32).reshape(1, _N)
  last = _NB - 1
  alpha = pl.pallas_call(
      _body,
      grid=(_NB + 1,),
      in_specs=[
          pl.BlockSpec((_BN, 93), lambda i: (jnp.minimum(i, last), 0)),
          pl.BlockSpec((1, _BN), lambda i: (0, jnp.minimum(i, last))),
          pl.BlockSpec((1, _N), lambda i: (0, 0)),
          pl.BlockSpec((16, 500), lambda i: (0, 0)),
          pl.BlockSpec((500, 150), lambda i: (0, 0)),
          pl.BlockSpec((1, 150), lambda i: (0, 0)),
          pl.BlockSpec((93, 150), lambda i: (0, 0)),
          pl.BlockSpec((1, 150), lambda i: (0, 0)),
          pl.BlockSpec((300, 150), lambda i: (0, 0)),
          pl.BlockSpec((150, 1), lambda i: (0, 0)),
          pl.BlockSpec((150, 1), lambda i: (0, 0)),
          pl.BlockSpec((1, 1), lambda i: (0, 0)),
      ],
      out_specs=pl.BlockSpec((1, _N), lambda i: (0, 0)),
      out_shape=jax.ShapeDtypeStruct((1, _N), jnp.float32),
      scratch_shapes=[
          pltpu.VMEM((150, 93), jnp.float32),
          pltpu.VMEM((150, _B), jnp.float32),
          pltpu.VMEM((_B, 1), jnp.float32),
          pltpu.VMEM((_B, 1), jnp.float32),
          pltpu.VMEM((_NB, _BN), jnp.float32),
      ],
  )(x_t, ids, ids, smile_latent, W_U1, b_U1.reshape(1, 150), W_U2,
    b_U2.reshape(1, 150), W_W1, b_W1.reshape(150, 1), W_W2,
    b_W2.reshape(1, 1))
  return alpha.reshape(_N, 1)


# speedup vs baseline: 1.2394x; 1.2394x over previous
import jax
import jax.numpy as jnp
from jax import lax
from jax.experimental import pallas as pl
from jax.experimental.pallas import tpu as pltpu

_N = 32768
_B = 16
_BN = 8192
_NB = _N // _BN
_NEG = -1.0e30


def _dotg(a, b, dims):
  return lax.dot_general(a, b, (dims, ((), ())),
                         preferred_element_type=jnp.float32)


def _body(x_ref, ids_ref, idsf_ref, sl_ref, wu1_ref, bu1_ref, wu2_ref,
          bu2_ref, ww1_ref, bw1_ref, ww2_ref, bw2_ref,
          a_ref, wxt_ref, ct_ref, m_ref, s_ref, e_ref):
  i = pl.program_id(0)

  @pl.when(i == 0)
  def _init():
    ww1 = ww1_ref[...]
    w1_top = ww1[:150, :]
    w1_bot = ww1[150:, :]
    wxt_ref[...] = _dotg(w1_bot, wu2_ref[...], ((0,), (1,)))
    u1 = jnp.dot(sl_ref[...], wu1_ref[...],
                 preferred_element_type=jnp.float32) + bu1_ref[...]
    ct_ref[...] = (_dotg(w1_top, u1, ((0,), (1,)))
                   + _dotg(w1_bot, bu2_ref[...], ((0,), (1,)))
                   + bw1_ref[...])
    m_ref[...] = jnp.full((_B, 1), _NEG, jnp.float32)
    s_ref[...] = jnp.zeros((_B, 1), jnp.float32)

  @pl.when(i < _NB)
  def _scores():
    ids = ids_ref[...]
    ohb = ids == lax.broadcasted_iota(jnp.int32, (_B, 1), 0)
    pre = _dotg(wxt_ref[...], x_ref[...], ((1,), (1,)))
    cg = _dotg(ct_ref[...], ohb.astype(jnp.float32), ((1,), (0,)))
    ht = jnp.tanh(pre + cg)
    et = jnp.sum(ht * ww2_ref[...], axis=0, keepdims=True) + bw2_ref[...]
    e_ref[pl.ds(i, 1), :] = et
    m_part = jnp.max(jnp.where(ohb, et, _NEG), axis=1, keepdims=True)
    s_part = jnp.sum(jnp.where(ohb, jnp.exp(et), 0.0), axis=1, keepdims=True)
    m_ref[...] = jnp.maximum(m_ref[...], m_part)
    s_ref[...] = s_ref[...] + s_part

  @pl.when(i == _NB)
  def _normalize():
    m = jnp.maximum(m_ref[...], -80.0)
    em = jnp.exp(-m)
    s = s_ref[...]
    q = jnp.where(s > 0.0, em / (em * s + 1e-16), 0.0)
    iota_b = lax.broadcasted_iota(jnp.int32, (_B, 1), 0)
    for j in range(_NB):
      ids_j = idsf_ref[0:1, j * _BN:(j + 1) * _BN]
      ohf = (ids_j == iota_b).astype(jnp.float32)
      qg = _dotg(q, ohf, ((0,), (0,)))
      a_ref[0:1, j * _BN:(j + 1) * _BN] = (
          jnp.exp(e_ref[j:j + 1, :]) * qg)


def kernel(x_t, x_t_batch, smile_latent, W_U1, b_U1, W_U2, b_U2,
           W_W1, b_W1, W_W2, b_W2):
  ids = x_t_batch.astype(jnp.int32).reshape(1, _N)
  last = _NB - 1
  alpha = pl.pallas_call(
      _body,
      grid=(_NB + 1,),
      in_specs=[
          pl.BlockSpec((_BN, 93), lambda i: (jnp.minimum(i, last), 0)),
          pl.BlockSpec((1, _BN), lambda i: (0, jnp.minimum(i, last))),
          pl.BlockSpec((1, _N), lambda i: (0, 0)),
          pl.BlockSpec((16, 500), lambda i: (0, 0)),
          pl.BlockSpec((500, 150), lambda i: (0, 0)),
          pl.BlockSpec((1, 150), lambda i: (0, 0)),
          pl.BlockSpec((93, 150), lambda i: (0, 0)),
          pl.BlockSpec((1, 150), lambda i: (0, 0)),
          pl.BlockSpec((300, 150), lambda i: (0, 0)),
          pl.BlockSpec((150, 1), lambda i: (0, 0)),
          pl.BlockSpec((150, 1), lambda i: (0, 0)),
          pl.BlockSpec((1, 1), lambda i: (0, 0)),
      ],
      out_specs=pl.BlockSpec((1, _N), lambda i: (0, 0)),
      out_shape=jax.ShapeDtypeStruct((1, _N), jnp.float32),
      scratch_shapes=[
          pltpu.VMEM((150, 93), jnp.float32),
          pltpu.VMEM((150, _B), jnp.float32),
          pltpu.VMEM((_B, 1), jnp.float32),
          pltpu.VMEM((_B, 1), jnp.float32),
          pltpu.VMEM((_NB, _BN), jnp.float32),
      ],
  )(x_t, ids, ids, smile_latent, W_U1, b_U1.reshape(1, 150), W_U2,
    b_U2.reshape(1, 150), W_W1, b_W1.reshape(150, 1), W_W2,
    b_W2.reshape(1, 1))
  return alpha.reshape(_N, 1)
